# Initial kernel scaffold; baseline (speedup 1.0000x reference)
import jax
import jax.numpy as jnp
from jax.experimental import pallas as pl


def kernel(x, edge_index, edge_attr, batch, graph_attr, W0, b0, We1, be1, We2, be2, Wroot, bconv, gru_Wih, gru_Whh, gru_bih, gru_bhh, lstm_Wih, lstm_Whh, lstm_bih, lstm_bhh, W1, b1, W2, b2):
    raise NotImplementedError("write your pallas kernel here")


# speedup vs baseline: 1.0004x; 1.0004x over previous
import functools

import jax
import jax.numpy as jnp
from jax import lax
from jax.experimental import pallas as pl
from jax.experimental.pallas import tpu as pltpu
from jax.experimental.pallas import tpu_sc as plsc

_NC = 2
_NS = 16
_NW = _NC * _NS
_CHUNK = 128

_MP_TIMES = 3
_STEPS = 3


def _sc_mesh():
    return plsc.VectorSubcoreMesh(core_axis_name="c", subcore_axis_name="s",
                                  num_cores=_NC, num_subcores=_NS)


@functools.partial(jax.jit, static_argnums=(2,))
def _sc_gather(idx_r, table, e_pad):
    n, d = table.shape
    ew = e_pad // _NW
    ch = ew // _CHUNK

    @functools.partial(
        pl.kernel,
        mesh=_sc_mesh(),
        compiler_params=pltpu.CompilerParams(use_tc_tiling_on_sc=False),
        out_type=jax.ShapeDtypeStruct((e_pad, d), jnp.float32),
        scratch_types=[
            pltpu.VMEM((ch, _CHUNK), jnp.int32),
            pltpu.VMEM((_CHUNK, d), jnp.float32),
            pltpu.VMEM((_CHUNK, d), jnp.float32),
            pltpu.SemaphoreType.DMA,
            pltpu.SemaphoreType.DMA,
        ],
    )
    def gather(idx_hbm, tab_hbm, xs_hbm, idxv, buf0, buf1, sem0, sem1):
        w = lax.axis_index("s") * _NC + lax.axis_index("c")
        base = w * ew
        pltpu.sync_copy(idx_hbm.at[w], idxv)
        pltpu.async_copy(tab_hbm.at[idxv.at[0]], buf0, sem0)

        @pl.loop(0, ch // 2)
        def _(i):
            j = i * 2
            pltpu.async_copy(tab_hbm.at[idxv.at[j + 1]], buf1, sem1)
            pltpu.make_async_copy(tab_hbm.at[idxv.at[j]], buf0, sem0).wait()
            pltpu.sync_copy(buf0, xs_hbm.at[pl.ds(base + j * _CHUNK, _CHUNK)])

            @pl.when(i < ch // 2 - 1)
            def _():
                pltpu.async_copy(tab_hbm.at[idxv.at[j + 2]], buf0, sem0)

            pltpu.make_async_copy(tab_hbm.at[idxv.at[j + 1]], buf1, sem1).wait()
            pltpu.sync_copy(
                buf1, xs_hbm.at[pl.ds(base + (j + 1) * _CHUNK, _CHUNK)])

    return gather(idx_r, table)


@functools.partial(jax.jit, static_argnums=(3,))
def _sc_scatter_add(idx_f, vals, zeros_nd, n):
    e_pad, d = vals.shape
    ew = e_pad // _NW
    ch = ew // _CHUNK
    rpt = n // _NS

    @functools.partial(
        pl.kernel,
        mesh=_sc_mesh(),
        compiler_params=pltpu.CompilerParams(use_tc_tiling_on_sc=False),
        out_type=jax.ShapeDtypeStruct((_NC * n, d), jnp.float32),
        scratch_types=[
            pltpu.VMEM((_CHUNK,), jnp.int32),
            pltpu.VMEM((_CHUNK, d), jnp.float32),
            pltpu.VMEM_SHARED((n, d), jnp.float32),
        ],
    )
    def scatter(idx_hbm, val_hbm, zer_hbm, out_hbm, idxc, buf, acc):
        cid = lax.axis_index("c")
        sid = lax.axis_index("s")
        w = sid * _NC + cid
        base = w * ew
        r0 = sid * rpt
        pltpu.sync_copy(zer_hbm.at[pl.ds(r0, rpt)], acc.at[pl.ds(r0, rpt)])
        plsc.subcore_barrier()

        @pl.loop(0, ch)
        def _(j):
            pltpu.sync_copy(idx_hbm.at[pl.ds(base + j * _CHUNK, _CHUNK)], idxc)
            pltpu.sync_copy(val_hbm.at[pl.ds(base + j * _CHUNK, _CHUNK)], buf)
            pltpu.sync_copy(buf, acc.at[idxc], add=True)

        plsc.subcore_barrier()
        pltpu.sync_copy(acc.at[pl.ds(r0, rpt)],
                        out_hbm.at[pl.ds(cid * n + r0, rpt)])

    return scatter(idx_f, vals, zeros_nd)


def _lin_relu_body(x_ref, w_ref, b_ref, o_ref):
    o_ref[...] = jnp.maximum(
        jnp.dot(x_ref[...], w_ref[...], preferred_element_type=jnp.float32)
        + b_ref[...], 0.0)


def _tc_lin_relu(xx, w, b):
    return pl.pallas_call(
        _lin_relu_body,
        out_shape=jax.ShapeDtypeStruct((xx.shape[0], w.shape[1]), jnp.float32),
    )(xx, w, b.reshape(1, -1))


def _tc_hidden(ea_p, we1, be1, blk=8192):
    e_pad = ea_p.shape[0]
    din, dh = we1.shape
    return pl.pallas_call(
        _lin_relu_body,
        grid=(e_pad // blk,),
        in_specs=[
            pl.BlockSpec((blk, din), lambda i: (i, 0)),
            pl.BlockSpec((din, dh), lambda i: (0, 0)),
            pl.BlockSpec((1, dh), lambda i: (0, 0)),
        ],
        out_specs=pl.BlockSpec((blk, dh), lambda i: (i, 0)),
        out_shape=jax.ShapeDtypeStruct((e_pad, dh), jnp.float32),
    )(ea_p, we1, be1.reshape(1, -1))


def _msg_body(e_total, blk, dc, hb_ref, xs_ref, w2_ref, b2_ref, o_ref):
    ewb = jnp.dot(hb_ref[...], w2_ref[...],
                  preferred_element_type=jnp.float32) + b2_ref[...]
    xs = xs_ref[...]
    acc = xs[:, 0:1] * ewb[:, 0:dc]
    for i in range(1, dc):
        acc = acc + xs[:, i:i + 1] * ewb[:, i * dc:(i + 1) * dc]
    rid = pl.program_id(0) * blk + lax.broadcasted_iota(jnp.int32, (blk, 1), 0)
    o_ref[...] = jnp.where(rid < e_total, acc, 0.0)


def _tc_msg(hidden, xs, we2, be2, e_total, blk=1024):
    e_pad = hidden.shape[0]
    dh = hidden.shape[1]
    dc = xs.shape[1]
    return pl.pallas_call(
        functools.partial(_msg_body, e_total, blk, dc),
        grid=(e_pad // blk,),
        in_specs=[
            pl.BlockSpec((blk, dh), lambda i: (i, 0)),
            pl.BlockSpec((blk, dc), lambda i: (i, 0)),
            pl.BlockSpec((dh, dc * dc), lambda i: (0, 0)),
            pl.BlockSpec((1, dc * dc), lambda i: (0, 0)),
        ],
        out_specs=pl.BlockSpec((blk, dc), lambda i: (i, 0)),
        out_shape=jax.ShapeDtypeStruct((e_pad, dc), jnp.float32),
    )(hidden, xs, we2, be2.reshape(1, -1))


def _update_body(n, dc, s_ref, aggp_ref, degp_ref, wr_ref, bc_ref,
                 wih_ref, whh_ref, bih_ref, bhh_ref, o_ref):
    s = s_ref[...]
    aggp = aggp_ref[...]
    degp = degp_ref[...]
    agg = aggp[0:n] + aggp[n:]
    deg = jnp.maximum(degp[0:n, 0:1] + degp[n:, 0:1], 1.0)
    m = jnp.maximum(
        jnp.dot(s, wr_ref[...], preferred_element_type=jnp.float32)
        + agg / deg + bc_ref[...], 0.0)
    gi = lax.dot_general(m, wih_ref[...], (((1,), (1,)), ((), ())),
                         preferred_element_type=jnp.float32) + bih_ref[...]
    gh = lax.dot_general(s, whh_ref[...], (((1,), (1,)), ((), ())),
                         preferred_element_type=jnp.float32) + bhh_ref[...]
    r = jax.nn.sigmoid(gi[:, 0:dc] + gh[:, 0:dc])
    z = jax.nn.sigmoid(gi[:, dc:2 * dc] + gh[:, dc:2 * dc])
    ng = jnp.tanh(gi[:, 2 * dc:3 * dc] + r * gh[:, 2 * dc:3 * dc])
    o_ref[...] = (1.0 - z) * ng + z * s


def _tc_update(s, aggp, degp, wroot, bconv, gwih, gwhh, gbih, gbhh):
    n, dc = s.shape
    return pl.pallas_call(
        functools.partial(_update_body, n, dc),
        out_shape=jax.ShapeDtypeStruct((n, dc), jnp.float32),
    )(s, aggp, degp, wroot, bconv.reshape(1, -1), gwih, gwhh,
      gbih.reshape(1, -1), gbhh.reshape(1, -1))


def _set2set_body(steps, ngr, n, dc, s_ref, batch_ref, ga_ref,
                  wih_ref, whh_ref, bih_ref, bhh_ref,
                  w1_ref, b1_ref, w2_ref, b2_ref, o_ref):
    s = s_ref[...]
    br = batch_ref[...]
    gidx = lax.broadcasted_iota(jnp.int32, (ngr, n), 0)
    msk = gidx == br
    qh = jnp.zeros((ngr, dc), jnp.float32)
    qc = jnp.zeros((ngr, dc), jnp.float32)
    q_star = jnp.zeros((ngr, 2 * dc), jnp.float32)
    for _ in range(steps):
        gates = (
            lax.dot_general(q_star, wih_ref[...], (((1,), (1,)), ((), ())),
                            preferred_element_type=jnp.float32)
            + bih_ref[...]
            + lax.dot_general(qh, whh_ref[...], (((1,), (1,)), ((), ())),
                              preferred_element_type=jnp.float32)
            + bhh_ref[...])
        ig = jax.nn.sigmoid(gates[:, 0:dc])
        fg = jax.nn.sigmoid(gates[:, dc:2 * dc])
        gg = jnp.tanh(gates[:, 2 * dc:3 * dc])
        og = jax.nn.sigmoid(gates[:, 3 * dc:4 * dc])
        qc = fg * qc + ig * gg
        qh = og * jnp.tanh(qc)
        qs = lax.dot_general(qh, s, (((1,), (1,)), ((), ())),
                             preferred_element_type=jnp.float32)
        eb = jnp.where(msk, qs, -jnp.inf)
        emax = jnp.max(eb, axis=1, keepdims=True)
        emax = jnp.where(emax > -jnp.inf, emax, 0.0)
        eeb = jnp.where(msk, jnp.exp(qs - emax), 0.0)
        den = jnp.sum(eeb, axis=1, keepdims=True)
        ab = eeb / (den + 1e-16)
        rvec = lax.dot_general(ab, s, (((1,), (0,)), ((), ())),
                               preferred_element_type=jnp.float32)
        q_star = jnp.concatenate([qh, rvec], axis=1)
    og2 = jnp.concatenate([q_star, ga_ref[...]], axis=1)
    h1 = jnp.maximum(
        lax.dot_general(og2, w1_ref[...], (((1,), (0,)), ((), ())),
                        preferred_element_type=jnp.float32) + b1_ref[...], 0.0)
    o_ref[...] = lax.dot_general(h1, w2_ref[...], (((1,), (0,)), ((), ())),
                                 preferred_element_type=jnp.float32) + b2_ref[...]


def _tc_set2set(s, batch_row, ga, lwih, lwhh, lbih, lbhh, w1, b1, w2, b2):
    n, dc = s.shape
    ngr = ga.shape[0]
    dout = w2.shape[1]
    return pl.pallas_call(
        functools.partial(_set2set_body, _STEPS, ngr, n, dc),
        out_shape=jax.ShapeDtypeStruct((ngr, dout), jnp.float32),
    )(s, batch_row, ga, lwih, lwhh, lbih.reshape(1, -1), lbhh.reshape(1, -1),
      w1, b1.reshape(1, -1), w2, b2.reshape(1, -1))


def kernel(x, edge_index, edge_attr, batch, graph_attr, W0, b0, We1, be1,
           We2, be2, Wroot, bconv, gru_Wih, gru_Whh, gru_bih, gru_bhh,
           lstm_Wih, lstm_Whh, lstm_bih, lstm_bhh, W1, b1, W2, b2):
    n = x.shape[0]
    e = edge_index.shape[1]
    dc = W0.shape[1]

    align = _NW * _CHUNK
    e_pad = ((e + align - 1) // align) * align
    ch = e_pad // _NW // _CHUNK
    pad = e_pad - e

    src_r = jnp.pad(edge_index[0], (0, pad)).reshape(_NW, ch, _CHUNK)
    dst_f = jnp.pad(edge_index[1], (0, pad))
    ea_p = jnp.pad(edge_attr, ((0, pad), (0, 0)))
    ones8 = jnp.pad(jnp.ones((e, 8), jnp.float32), ((0, pad), (0, 0)))
    zer_dc = jnp.zeros((n, dc), jnp.float32)
    zer8 = jnp.zeros((n, 8), jnp.float32)
    batch_row = batch.reshape(1, n)

    s = _tc_lin_relu(x, W0, b0)
    hidden = _tc_hidden(ea_p, We1, be1)
    degp = _sc_scatter_add(dst_f, ones8, zer8, n)

    for _ in range(_MP_TIMES):
        xs = _sc_gather(src_r, s, e_pad)
        msg = _tc_msg(hidden, xs, We2, be2, e)
        aggp = _sc_scatter_add(dst_f, msg, zer_dc, n)
        s = _tc_update(s, aggp, degp, Wroot, bconv,
                       gru_Wih, gru_Whh, gru_bih, gru_bhh)

    return _tc_set2set(s, batch_row, graph_attr, lstm_Wih, lstm_Whh,
                       lstm_bih, lstm_bhh, W1, b1, W2, b2)


# speedup vs baseline: 2.4404x; 2.4394x over previous
import functools

import jax
import jax.numpy as jnp
from jax import lax
from jax.experimental import pallas as pl
from jax.experimental.pallas import tpu as pltpu
from jax.experimental.pallas import tpu_sc as plsc

_NC = 2
_NS = 16
_NW = _NC * _NS
_CHUNK = 128

_MP_TIMES = 3
_STEPS = 3


def _sc_mesh():
    return plsc.VectorSubcoreMesh(core_axis_name="c", subcore_axis_name="s",
                                  num_cores=_NC, num_subcores=_NS)


@functools.partial(jax.jit, static_argnums=(2,))
def _sc_gather(idx_r, table, e_pad):
    n, d = table.shape
    ew = e_pad // _NW
    ch = ew // _CHUNK

    @functools.partial(
        pl.kernel,
        mesh=_sc_mesh(),
        compiler_params=pltpu.CompilerParams(use_tc_tiling_on_sc=False),
        out_type=jax.ShapeDtypeStruct((e_pad, d), jnp.float32),
        scratch_types=[
            pltpu.VMEM((ch, _CHUNK), jnp.int32),
            pltpu.VMEM((_CHUNK, d), jnp.float32),
            pltpu.VMEM((_CHUNK, d), jnp.float32),
            pltpu.SemaphoreType.DMA,
            pltpu.SemaphoreType.DMA,
        ],
    )
    def gather(idx_hbm, tab_hbm, xs_hbm, idxv, buf0, buf1, sem0, sem1):
        w = lax.axis_index("s") * _NC + lax.axis_index("c")
        base = w * ew
        pltpu.sync_copy(idx_hbm.at[w], idxv)
        pltpu.async_copy(tab_hbm.at[idxv.at[0]], buf0, sem0)

        @pl.loop(0, ch // 2)
        def _(i):
            j = i * 2
            pltpu.async_copy(tab_hbm.at[idxv.at[j + 1]], buf1, sem1)
            pltpu.make_async_copy(tab_hbm.at[idxv.at[j]], buf0, sem0).wait()
            pltpu.sync_copy(buf0, xs_hbm.at[pl.ds(base + j * _CHUNK, _CHUNK)])

            @pl.when(i < ch // 2 - 1)
            def _():
                pltpu.async_copy(tab_hbm.at[idxv.at[j + 2]], buf0, sem0)

            pltpu.make_async_copy(tab_hbm.at[idxv.at[j + 1]], buf1, sem1).wait()
            pltpu.sync_copy(
                buf1, xs_hbm.at[pl.ds(base + (j + 1) * _CHUNK, _CHUNK)])

    return gather(idx_r, table)


@functools.partial(jax.jit, static_argnums=(3,))
def _sc_scatter_add(idx_f, vals, zeros_nd, n):
    e_pad, d = vals.shape
    ew = e_pad // _NW
    ch = ew // _CHUNK
    rpt = n // _NS

    @functools.partial(
        pl.kernel,
        mesh=_sc_mesh(),
        compiler_params=pltpu.CompilerParams(use_tc_tiling_on_sc=False),
        out_type=jax.ShapeDtypeStruct((_NC * n, d), jnp.float32),
        scratch_types=[
            pltpu.VMEM((_CHUNK,), jnp.int32),
            pltpu.VMEM((_CHUNK,), jnp.int32),
            pltpu.VMEM((_CHUNK, d), jnp.float32),
            pltpu.VMEM((_CHUNK, d), jnp.float32),
            pltpu.VMEM_SHARED((n, d), jnp.float32),
            pltpu.SemaphoreType.DMA,
            pltpu.SemaphoreType.DMA,
        ],
    )
    def scatter(idx_hbm, val_hbm, zer_hbm, out_hbm, idx0, idx1, buf0, buf1,
                acc, sem0, sem1):
        cid = lax.axis_index("c")
        sid = lax.axis_index("s")
        w = sid * _NC + cid
        base = w * ew
        r0 = sid * rpt
        pltpu.sync_copy(zer_hbm.at[pl.ds(r0, rpt)], acc.at[pl.ds(r0, rpt)])
        plsc.subcore_barrier()

        @pl.loop(0, ch // 2)
        def _(i):
            j = i * 2

            @pl.when(i > 0)
            def _():
                pltpu.make_async_copy(buf0, acc.at[idx0], sem0).wait()
            pltpu.sync_copy(idx_hbm.at[pl.ds(base + j * _CHUNK, _CHUNK)], idx0)
            pltpu.sync_copy(val_hbm.at[pl.ds(base + j * _CHUNK, _CHUNK)], buf0)
            pltpu.async_copy(buf0, acc.at[idx0], sem0, add=True)

            @pl.when(i > 0)
            def _():
                pltpu.make_async_copy(buf1, acc.at[idx1], sem1).wait()
            pltpu.sync_copy(
                idx_hbm.at[pl.ds(base + (j + 1) * _CHUNK, _CHUNK)], idx1)
            pltpu.sync_copy(
                val_hbm.at[pl.ds(base + (j + 1) * _CHUNK, _CHUNK)], buf1)
            pltpu.async_copy(buf1, acc.at[idx1], sem1, add=True)

        pltpu.make_async_copy(buf0, acc.at[idx0], sem0).wait()
        pltpu.make_async_copy(buf1, acc.at[idx1], sem1).wait()
        plsc.subcore_barrier()
        pltpu.sync_copy(acc.at[pl.ds(r0, rpt)],
                        out_hbm.at[pl.ds(cid * n + r0, rpt)])

    return scatter(idx_f, vals, zeros_nd)


def _lin_relu_body(x_ref, w_ref, b_ref, o_ref):
    o_ref[...] = jnp.maximum(
        jnp.dot(x_ref[...], w_ref[...], preferred_element_type=jnp.float32)
        + b_ref[...], 0.0)


def _tc_lin_relu(xx, w, b):
    return pl.pallas_call(
        _lin_relu_body,
        out_shape=jax.ShapeDtypeStruct((xx.shape[0], w.shape[1]), jnp.float32),
    )(xx, w, b.reshape(1, -1))


def _tc_hidden(ea_p, we1, be1, blk=8192):
    e_pad = ea_p.shape[0]
    din, dh = we1.shape
    return pl.pallas_call(
        _lin_relu_body,
        grid=(e_pad // blk,),
        in_specs=[
            pl.BlockSpec((blk, din), lambda i: (i, 0)),
            pl.BlockSpec((din, dh), lambda i: (0, 0)),
            pl.BlockSpec((1, dh), lambda i: (0, 0)),
        ],
        out_specs=pl.BlockSpec((blk, dh), lambda i: (i, 0)),
        out_shape=jax.ShapeDtypeStruct((e_pad, dh), jnp.float32),
    )(ea_p, we1, be1.reshape(1, -1))


def _msg_body(e_total, blk, hb_ref, xs_ref, w2_ref, b2_ref, rexp_ref,
              rsum_ref, o_ref):
    ewb = jnp.dot(hb_ref[...], w2_ref[...],
                  preferred_element_type=jnp.float32) + b2_ref[...]
    xse = jnp.dot(xs_ref[...], rexp_ref[...],
                  preferred_element_type=jnp.float32)
    acc = jnp.dot(ewb * xse, rsum_ref[...],
                  preferred_element_type=jnp.float32)
    rid = pl.program_id(0) * blk + lax.broadcasted_iota(jnp.int32, (blk, 1), 0)
    o_ref[...] = jnp.where(rid < e_total, acc, 0.0)


def _tc_msg(hidden, xs, we2, be2, rexp, rsum, e_total, blk=1024):
    e_pad = hidden.shape[0]
    dh = hidden.shape[1]
    dc = xs.shape[1]
    return pl.pallas_call(
        functools.partial(_msg_body, e_total, blk),
        grid=(e_pad // blk,),
        in_specs=[
            pl.BlockSpec((blk, dh), lambda i: (i, 0)),
            pl.BlockSpec((blk, dc), lambda i: (i, 0)),
            pl.BlockSpec((dh, dc * dc), lambda i: (0, 0)),
            pl.BlockSpec((1, dc * dc), lambda i: (0, 0)),
            pl.BlockSpec((dc, dc * dc), lambda i: (0, 0)),
            pl.BlockSpec((dc * dc, dc), lambda i: (0, 0)),
        ],
        out_specs=pl.BlockSpec((blk, dc), lambda i: (i, 0)),
        out_shape=jax.ShapeDtypeStruct((e_pad, dc), jnp.float32),
    )(hidden, xs, we2, be2.reshape(1, -1), rexp, rsum)


def _update_body(n, dc, s_ref, aggp_ref, degp_ref, wr_ref, bc_ref,
                 wih_ref, whh_ref, bih_ref, bhh_ref, o_ref):
    s = s_ref[...]
    aggp = aggp_ref[...]
    degp = degp_ref[...]
    agg = aggp[0:n] + aggp[n:]
    deg = jnp.maximum(degp[0:n, 0:1] + degp[n:, 0:1], 1.0)
    m = jnp.maximum(
        jnp.dot(s, wr_ref[...], preferred_element_type=jnp.float32)
        + agg / deg + bc_ref[...], 0.0)
    gi = lax.dot_general(m, wih_ref[...], (((1,), (1,)), ((), ())),
                         preferred_element_type=jnp.float32) + bih_ref[...]
    gh = lax.dot_general(s, whh_ref[...], (((1,), (1,)), ((), ())),
                         preferred_element_type=jnp.float32) + bhh_ref[...]
    r = jax.nn.sigmoid(gi[:, 0:dc] + gh[:, 0:dc])
    z = jax.nn.sigmoid(gi[:, dc:2 * dc] + gh[:, dc:2 * dc])
    ng = jnp.tanh(gi[:, 2 * dc:3 * dc] + r * gh[:, 2 * dc:3 * dc])
    o_ref[...] = (1.0 - z) * ng + z * s


def _tc_update(s, aggp, degp, wroot, bconv, gwih, gwhh, gbih, gbhh):
    n, dc = s.shape
    return pl.pallas_call(
        functools.partial(_update_body, n, dc),
        out_shape=jax.ShapeDtypeStruct((n, dc), jnp.float32),
    )(s, aggp, degp, wroot, bconv.reshape(1, -1), gwih, gwhh,
      gbih.reshape(1, -1), gbhh.reshape(1, -1))


def _set2set_body(steps, ngr, n, dc, s_ref, batch_ref, ga_ref,
                  wih_ref, whh_ref, bih_ref, bhh_ref,
                  w1_ref, b1_ref, w2_ref, b2_ref, o_ref):
    s = s_ref[...]
    br = batch_ref[...]
    gidx = lax.broadcasted_iota(jnp.int32, (ngr, n), 0)
    msk = gidx == br
    qh = jnp.zeros((ngr, dc), jnp.float32)
    qc = jnp.zeros((ngr, dc), jnp.float32)
    q_star = jnp.zeros((ngr, 2 * dc), jnp.float32)
    for _ in range(steps):
        gates = (
            lax.dot_general(q_star, wih_ref[...], (((1,), (1,)), ((), ())),
                            preferred_element_type=jnp.float32)
            + bih_ref[...]
            + lax.dot_general(qh, whh_ref[...], (((1,), (1,)), ((), ())),
                              preferred_element_type=jnp.float32)
            + bhh_ref[...])
        ig = jax.nn.sigmoid(gates[:, 0:dc])
        fg = jax.nn.sigmoid(gates[:, dc:2 * dc])
        gg = jnp.tanh(gates[:, 2 * dc:3 * dc])
        og = jax.nn.sigmoid(gates[:, 3 * dc:4 * dc])
        qc = fg * qc + ig * gg
        qh = og * jnp.tanh(qc)
        qs = lax.dot_general(qh, s, (((1,), (1,)), ((), ())),
                             preferred_element_type=jnp.float32)
        eb = jnp.where(msk, qs, -jnp.inf)
        emax = jnp.max(eb, axis=1, keepdims=True)
        emax = jnp.where(emax > -jnp.inf, emax, 0.0)
        eeb = jnp.where(msk, jnp.exp(qs - emax), 0.0)
        den = jnp.sum(eeb, axis=1, keepdims=True)
        ab = eeb / (den + 1e-16)
        rvec = lax.dot_general(ab, s, (((1,), (0,)), ((), ())),
                               preferred_element_type=jnp.float32)
        q_star = jnp.concatenate([qh, rvec], axis=1)
    og2 = jnp.concatenate([q_star, ga_ref[...]], axis=1)
    h1 = jnp.maximum(
        lax.dot_general(og2, w1_ref[...], (((1,), (0,)), ((), ())),
                        preferred_element_type=jnp.float32) + b1_ref[...], 0.0)
    o_ref[...] = lax.dot_general(h1, w2_ref[...], (((1,), (0,)), ((), ())),
                                 preferred_element_type=jnp.float32) + b2_ref[...]


def _tc_set2set(s, batch_row, ga, lwih, lwhh, lbih, lbhh, w1, b1, w2, b2):
    n, dc = s.shape
    ngr = ga.shape[0]
    dout = w2.shape[1]
    return pl.pallas_call(
        functools.partial(_set2set_body, _STEPS, ngr, n, dc),
        out_shape=jax.ShapeDtypeStruct((ngr, dout), jnp.float32),
    )(s, batch_row, ga, lwih, lwhh, lbih.reshape(1, -1), lbhh.reshape(1, -1),
      w1, b1.reshape(1, -1), w2, b2.reshape(1, -1))


def kernel(x, edge_index, edge_attr, batch, graph_attr, W0, b0, We1, be1,
           We2, be2, Wroot, bconv, gru_Wih, gru_Whh, gru_bih, gru_bhh,
           lstm_Wih, lstm_Whh, lstm_bih, lstm_bhh, W1, b1, W2, b2):
    n = x.shape[0]
    e = edge_index.shape[1]
    dc = W0.shape[1]

    align = _NW * _CHUNK
    e_pad = ((e + align - 1) // align) * align
    ch = e_pad // _NW // _CHUNK
    pad = e_pad - e

    src_r = jnp.pad(edge_index[0], (0, pad)).reshape(_NW, ch, _CHUNK)
    dst_f = jnp.pad(edge_index[1], (0, pad))
    ea_p = jnp.pad(edge_attr, ((0, pad), (0, 0)))
    ones8 = jnp.pad(jnp.ones((e, 8), jnp.float32), ((0, pad), (0, 0)))
    zer_dc = jnp.zeros((n, dc), jnp.float32)
    zer8 = jnp.zeros((n, 8), jnp.float32)
    batch_row = batch.reshape(1, n)

    s = _tc_lin_relu(x, W0, b0)
    hidden = _tc_hidden(ea_p, We1, be1)
    degp = _sc_scatter_add(dst_f, ones8, zer8, n)

    lane = jnp.arange(dc * dc, dtype=jnp.int32)
    rexp = (lane[None, :] // dc == jnp.arange(dc, dtype=jnp.int32)[:, None]
            ).astype(jnp.float32)
    rsum = (lane[:, None] % dc == jnp.arange(dc, dtype=jnp.int32)[None, :]
            ).astype(jnp.float32)

    for _ in range(_MP_TIMES):
        xs = _sc_gather(src_r, s, e_pad)
        msg = _tc_msg(hidden, xs, We2, be2, rexp, rsum, e)
        aggp = _sc_scatter_add(dst_f, msg, zer_dc, n)
        s = _tc_update(s, aggp, degp, Wroot, bconv,
                       gru_Wih, gru_Whh, gru_bih, gru_bhh)

    return _tc_set2set(s, batch_row, graph_attr, lstm_Wih, lstm_Whh,
                       lstm_bih, lstm_bhh, W1, b1, W2, b2)


# speedup vs baseline: 2.6716x; 1.0947x over previous
import functools

import jax
import jax.numpy as jnp
from jax import lax
from jax.experimental import pallas as pl
from jax.experimental.pallas import tpu as pltpu
from jax.experimental.pallas import tpu_sc as plsc

_NC = 2
_NS = 16
_NW = _NC * _NS
_CHUNK = 128

_MP_TIMES = 3
_STEPS = 3


def _sc_mesh():
    return plsc.VectorSubcoreMesh(core_axis_name="c", subcore_axis_name="s",
                                  num_cores=_NC, num_subcores=_NS)


@functools.partial(jax.jit, static_argnums=(2,))
def _sc_gather(idx_r, table, e_pad):
    n, d = table.shape
    ew = e_pad // _NW
    ch = ew // _CHUNK

    @functools.partial(
        pl.kernel,
        mesh=_sc_mesh(),
        compiler_params=pltpu.CompilerParams(use_tc_tiling_on_sc=False),
        out_type=jax.ShapeDtypeStruct((e_pad, d), jnp.bfloat16),
        scratch_types=[
            pltpu.VMEM((ch, _CHUNK), jnp.int32),
            pltpu.VMEM((_CHUNK, d), jnp.bfloat16),
            pltpu.VMEM((_CHUNK, d), jnp.bfloat16),
            pltpu.SemaphoreType.DMA,
            pltpu.SemaphoreType.DMA,
        ],
    )
    def gather(idx_hbm, tab_hbm, xs_hbm, idxv, buf0, buf1, sem0, sem1):
        w = lax.axis_index("s") * _NC + lax.axis_index("c")
        base = w * ew
        pltpu.sync_copy(idx_hbm.at[w], idxv)
        pltpu.async_copy(tab_hbm.at[idxv.at[0]], buf0, sem0)

        @pl.loop(0, ch // 2)
        def _(i):
            j = i * 2
            pltpu.async_copy(tab_hbm.at[idxv.at[j + 1]], buf1, sem1)
            pltpu.make_async_copy(tab_hbm.at[idxv.at[j]], buf0, sem0).wait()
            pltpu.sync_copy(buf0, xs_hbm.at[pl.ds(base + j * _CHUNK, _CHUNK)])

            @pl.when(i < ch // 2 - 1)
            def _():
                pltpu.async_copy(tab_hbm.at[idxv.at[j + 2]], buf0, sem0)

            pltpu.make_async_copy(tab_hbm.at[idxv.at[j + 1]], buf1, sem1).wait()
            pltpu.sync_copy(
                buf1, xs_hbm.at[pl.ds(base + (j + 1) * _CHUNK, _CHUNK)])

    return gather(idx_r, table)


@functools.partial(jax.jit, static_argnums=(3,))
def _sc_scatter_add(idx_f, vals, zeros_nd, n):
    e_pad, d = vals.shape
    ew = e_pad // _NW
    ch = ew // _CHUNK
    rpt = n // _NS

    @functools.partial(
        pl.kernel,
        mesh=_sc_mesh(),
        compiler_params=pltpu.CompilerParams(use_tc_tiling_on_sc=False),
        out_type=jax.ShapeDtypeStruct((_NC * n, d), jnp.float32),
        scratch_types=[
            pltpu.VMEM((_CHUNK,), jnp.int32),
            pltpu.VMEM((_CHUNK,), jnp.int32),
            pltpu.VMEM((_CHUNK, d), jnp.float32),
            pltpu.VMEM((_CHUNK, d), jnp.float32),
            pltpu.VMEM_SHARED((n, d), jnp.float32),
            pltpu.SemaphoreType.DMA,
            pltpu.SemaphoreType.DMA,
        ],
    )
    def scatter(idx_hbm, val_hbm, zer_hbm, out_hbm, idx0, idx1, buf0, buf1,
                acc, sem0, sem1):
        cid = lax.axis_index("c")
        sid = lax.axis_index("s")
        w = sid * _NC + cid
        base = w * ew
        r0 = sid * rpt
        pltpu.sync_copy(zer_hbm.at[pl.ds(r0, rpt)], acc.at[pl.ds(r0, rpt)])
        plsc.subcore_barrier()

        @pl.loop(0, ch // 2)
        def _(i):
            j = i * 2

            @pl.when(i > 0)
            def _():
                pltpu.make_async_copy(buf0, acc.at[idx0], sem0).wait()
            pltpu.sync_copy(idx_hbm.at[pl.ds(base + j * _CHUNK, _CHUNK)], idx0)
            pltpu.sync_copy(val_hbm.at[pl.ds(base + j * _CHUNK, _CHUNK)], buf0)
            pltpu.async_copy(buf0, acc.at[idx0], sem0, add=True)

            @pl.when(i > 0)
            def _():
                pltpu.make_async_copy(buf1, acc.at[idx1], sem1).wait()
            pltpu.sync_copy(
                idx_hbm.at[pl.ds(base + (j + 1) * _CHUNK, _CHUNK)], idx1)
            pltpu.sync_copy(
                val_hbm.at[pl.ds(base + (j + 1) * _CHUNK, _CHUNK)], buf1)
            pltpu.async_copy(buf1, acc.at[idx1], sem1, add=True)

        pltpu.make_async_copy(buf0, acc.at[idx0], sem0).wait()
        pltpu.make_async_copy(buf1, acc.at[idx1], sem1).wait()
        plsc.subcore_barrier()
        pltpu.sync_copy(acc.at[pl.ds(r0, rpt)],
                        out_hbm.at[pl.ds(cid * n + r0, rpt)])

    return scatter(idx_f, vals, zeros_nd)


def _lin_relu2_body(x_ref, w_ref, b_ref, o_ref, ob_ref):
    r = jnp.maximum(
        jnp.dot(x_ref[...], w_ref[...], preferred_element_type=jnp.float32)
        + b_ref[...], 0.0)
    o_ref[...] = r
    ob_ref[...] = r.astype(jnp.bfloat16)


def _tc_lin_relu2(xx, w, b):
    n, dc = xx.shape[0], w.shape[1]
    return pl.pallas_call(
        _lin_relu2_body,
        out_shape=[jax.ShapeDtypeStruct((n, dc), jnp.float32),
                   jax.ShapeDtypeStruct((n, dc), jnp.bfloat16)],
    )(xx, w, b.reshape(1, -1))


def _hidden_body(x_ref, w_ref, b_ref, o_ref):
    o_ref[...] = jnp.maximum(
        jnp.dot(x_ref[...], w_ref[...], preferred_element_type=jnp.float32)
        + b_ref[...], 0.0).astype(jnp.bfloat16)


def _tc_hidden(ea_p, we1, be1, blk=8192):
    e_pad = ea_p.shape[0]
    din, dh = we1.shape
    return pl.pallas_call(
        _hidden_body,
        grid=(e_pad // blk,),
        in_specs=[
            pl.BlockSpec((blk, din), lambda i: (i, 0)),
            pl.BlockSpec((din, dh), lambda i: (0, 0)),
            pl.BlockSpec((1, dh), lambda i: (0, 0)),
        ],
        out_specs=pl.BlockSpec((blk, dh), lambda i: (i, 0)),
        out_shape=jax.ShapeDtypeStruct((e_pad, dh), jnp.bfloat16),
    )(ea_p, we1, be1.reshape(1, -1))


def _msg_body(e_total, blk, hb_ref, xs_ref, w2_ref, bmat_ref, rexp_ref,
              rsum_ref, o_ref):
    ewb = jnp.dot(hb_ref[...], w2_ref[...],
                  preferred_element_type=jnp.float32)
    xse = jnp.dot(xs_ref[...], rexp_ref[...],
                  preferred_element_type=jnp.float32)
    acc = (jnp.dot((ewb * xse).astype(jnp.bfloat16), rsum_ref[...],
                   preferred_element_type=jnp.float32)
           + jnp.dot(xs_ref[...], bmat_ref[...],
                     preferred_element_type=jnp.float32))
    rid = pl.program_id(0) * blk + lax.broadcasted_iota(jnp.int32, (blk, 1), 0)
    o_ref[...] = jnp.where(rid < e_total, acc, 0.0)


def _tc_msg(hidden, xs, we2, bmat, rexp, rsum, e_total, blk=2048):
    e_pad = hidden.shape[0]
    dh = hidden.shape[1]
    dc = xs.shape[1]
    return pl.pallas_call(
        functools.partial(_msg_body, e_total, blk),
        grid=(e_pad // blk,),
        in_specs=[
            pl.BlockSpec((blk, dh), lambda i: (i, 0)),
            pl.BlockSpec((blk, dc), lambda i: (i, 0)),
            pl.BlockSpec((dh, dc * dc), lambda i: (0, 0)),
            pl.BlockSpec((dc, dc), lambda i: (0, 0)),
            pl.BlockSpec((dc, dc * dc), lambda i: (0, 0)),
            pl.BlockSpec((dc * dc, dc), lambda i: (0, 0)),
        ],
        out_specs=pl.BlockSpec((blk, dc), lambda i: (i, 0)),
        out_shape=jax.ShapeDtypeStruct((e_pad, dc), jnp.float32),
    )(hidden, xs, we2, bmat, rexp, rsum)


def _update_body(n, dc, s_ref, aggp_ref, degp_ref, wr_ref, bc_ref,
                 wih_ref, whh_ref, bih_ref, bhh_ref, o_ref, ob_ref):
    s = s_ref[...]
    aggp = aggp_ref[...]
    degp = degp_ref[...]
    agg = aggp[0:n] + aggp[n:]
    deg = jnp.maximum(degp[0:n, 0:1] + degp[n:, 0:1], 1.0)
    m = jnp.maximum(
        jnp.dot(s, wr_ref[...], preferred_element_type=jnp.float32)
        + agg / deg + bc_ref[...], 0.0)
    gi = lax.dot_general(m, wih_ref[...], (((1,), (1,)), ((), ())),
                         preferred_element_type=jnp.float32) + bih_ref[...]
    gh = lax.dot_general(s, whh_ref[...], (((1,), (1,)), ((), ())),
                         preferred_element_type=jnp.float32) + bhh_ref[...]
    r = jax.nn.sigmoid(gi[:, 0:dc] + gh[:, 0:dc])
    z = jax.nn.sigmoid(gi[:, dc:2 * dc] + gh[:, dc:2 * dc])
    ng = jnp.tanh(gi[:, 2 * dc:3 * dc] + r * gh[:, 2 * dc:3 * dc])
    s_new = (1.0 - z) * ng + z * s
    o_ref[...] = s_new
    ob_ref[...] = s_new.astype(jnp.bfloat16)


def _tc_update(s, aggp, degp, wroot, bconv, gwih, gwhh, gbih, gbhh):
    n, dc = s.shape
    return pl.pallas_call(
        functools.partial(_update_body, n, dc),
        out_shape=[jax.ShapeDtypeStruct((n, dc), jnp.float32),
                   jax.ShapeDtypeStruct((n, dc), jnp.bfloat16)],
    )(s, aggp, degp, wroot, bconv.reshape(1, -1), gwih, gwhh,
      gbih.reshape(1, -1), gbhh.reshape(1, -1))


def _set2set_body(steps, ngr, n, dc, s_ref, batch_ref, ga_ref,
                  wih_ref, whh_ref, bih_ref, bhh_ref,
                  w1_ref, b1_ref, w2_ref, b2_ref, o_ref):
    s = s_ref[...]
    br = batch_ref[...]
    gidx = lax.broadcasted_iota(jnp.int32, (ngr, n), 0)
    msk = gidx == br
    qh = jnp.zeros((ngr, dc), jnp.float32)
    qc = jnp.zeros((ngr, dc), jnp.float32)
    q_star = jnp.zeros((ngr, 2 * dc), jnp.float32)
    for _ in range(steps):
        gates = (
            lax.dot_general(q_star, wih_ref[...], (((1,), (1,)), ((), ())),
                            preferred_element_type=jnp.float32)
            + bih_ref[...]
            + lax.dot_general(qh, whh_ref[...], (((1,), (1,)), ((), ())),
                              preferred_element_type=jnp.float32)
            + bhh_ref[...])
        ig = jax.nn.sigmoid(gates[:, 0:dc])
        fg = jax.nn.sigmoid(gates[:, dc:2 * dc])
        gg = jnp.tanh(gates[:, 2 * dc:3 * dc])
        og = jax.nn.sigmoid(gates[:, 3 * dc:4 * dc])
        qc = fg * qc + ig * gg
        qh = og * jnp.tanh(qc)
        qs = lax.dot_general(qh, s, (((1,), (1,)), ((), ())),
                             preferred_element_type=jnp.float32)
        eb = jnp.where(msk, qs, -jnp.inf)
        emax = jnp.max(eb, axis=1, keepdims=True)
        emax = jnp.where(emax > -jnp.inf, emax, 0.0)
        eeb = jnp.where(msk, jnp.exp(qs - emax), 0.0)
        den = jnp.sum(eeb, axis=1, keepdims=True)
        ab = eeb / (den + 1e-16)
        rvec = lax.dot_general(ab, s, (((1,), (0,)), ((), ())),
                               preferred_element_type=jnp.float32)
        q_star = jnp.concatenate([qh, rvec], axis=1)
    og2 = jnp.concatenate([q_star, ga_ref[...]], axis=1)
    h1 = jnp.maximum(
        lax.dot_general(og2, w1_ref[...], (((1,), (0,)), ((), ())),
                        preferred_element_type=jnp.float32) + b1_ref[...], 0.0)
    o_ref[...] = lax.dot_general(h1, w2_ref[...], (((1,), (0,)), ((), ())),
                                 preferred_element_type=jnp.float32) + b2_ref[...]


def _tc_set2set(s, batch_row, ga, lwih, lwhh, lbih, lbhh, w1, b1, w2, b2):
    n, dc = s.shape
    ngr = ga.shape[0]
    dout = w2.shape[1]
    return pl.pallas_call(
        functools.partial(_set2set_body, _STEPS, ngr, n, dc),
        out_shape=jax.ShapeDtypeStruct((ngr, dout), jnp.float32),
    )(s, batch_row, ga, lwih, lwhh, lbih.reshape(1, -1), lbhh.reshape(1, -1),
      w1, b1.reshape(1, -1), w2, b2.reshape(1, -1))


def kernel(x, edge_index, edge_attr, batch, graph_attr, W0, b0, We1, be1,
           We2, be2, Wroot, bconv, gru_Wih, gru_Whh, gru_bih, gru_bhh,
           lstm_Wih, lstm_Whh, lstm_bih, lstm_bhh, W1, b1, W2, b2):
    n = x.shape[0]
    e = edge_index.shape[1]
    dc = W0.shape[1]

    align = _NW * _CHUNK
    e_pad = ((e + align - 1) // align) * align
    ch = e_pad // _NW // _CHUNK
    pad = e_pad - e

    src_r = jnp.pad(edge_index[0], (0, pad)).reshape(_NW, ch, _CHUNK)
    dst_f = jnp.pad(edge_index[1], (0, pad))
    ea_p = jnp.pad(edge_attr, ((0, pad), (0, 0)))
    ones8 = jnp.pad(jnp.ones((e, 8), jnp.float32), ((0, pad), (0, 0)))
    zer_dc = jnp.zeros((n, dc), jnp.float32)
    zer8 = jnp.zeros((n, 8), jnp.float32)
    batch_row = batch.reshape(1, n)

    s, s_bf = _tc_lin_relu2(x, W0, b0)
    hidden = _tc_hidden(ea_p, We1, be1)
    degp = _sc_scatter_add(dst_f, ones8, zer8, n)
    we2b = We2.astype(jnp.bfloat16)

    bmat = be2.reshape(dc, dc)
    lane = jnp.arange(dc * dc, dtype=jnp.int32)
    rexp = (lane[None, :] // dc == jnp.arange(dc, dtype=jnp.int32)[:, None]
            ).astype(jnp.bfloat16)
    rsum = (lane[:, None] % dc == jnp.arange(dc, dtype=jnp.int32)[None, :]
            ).astype(jnp.bfloat16)

    for _ in range(_MP_TIMES):
        xs = _sc_gather(src_r, s_bf, e_pad)
        msg = _tc_msg(hidden, xs, we2b, bmat, rexp, rsum, e)
        aggp = _sc_scatter_add(dst_f, msg, zer_dc, n)
        s, s_bf = _tc_update(s, aggp, degp, Wroot, bconv,
                             gru_Wih, gru_Whh, gru_bih, gru_bhh)

    return _tc_set2set(s, batch_row, graph_attr, lstm_Wih, lstm_Whh,
                       lstm_bih, lstm_bhh, W1, b1, W2, b2)


# speedup vs baseline: 2.7685x; 1.0363x over previous
import functools

import jax
import jax.numpy as jnp
from jax import lax
from jax.experimental import pallas as pl
from jax.experimental.pallas import tpu as pltpu
from jax.experimental.pallas import tpu_sc as plsc

_NC = 2
_NS = 16
_NW = _NC * _NS
_CHUNK = 128

_MP_TIMES = 3
_STEPS = 3


def _sc_mesh():
    return plsc.VectorSubcoreMesh(core_axis_name="c", subcore_axis_name="s",
                                  num_cores=_NC, num_subcores=_NS)


@functools.partial(jax.jit, static_argnums=(2,))
def _sc_gather(idx_r, table, e_pad):
    n, d = table.shape
    ew = e_pad // _NW
    ch = ew // _CHUNK

    @functools.partial(
        pl.kernel,
        mesh=_sc_mesh(),
        compiler_params=pltpu.CompilerParams(use_tc_tiling_on_sc=False),
        out_type=jax.ShapeDtypeStruct((e_pad, d), jnp.bfloat16),
        scratch_types=[
            pltpu.VMEM((ch, _CHUNK), jnp.int32),
            pltpu.VMEM((_CHUNK, d), jnp.bfloat16),
            pltpu.VMEM((_CHUNK, d), jnp.bfloat16),
            pltpu.SemaphoreType.DMA,
            pltpu.SemaphoreType.DMA,
        ],
    )
    def gather(idx_hbm, tab_hbm, xs_hbm, idxv, buf0, buf1, sem0, sem1):
        w = lax.axis_index("s") * _NC + lax.axis_index("c")
        base = w * ew
        pltpu.sync_copy(idx_hbm.at[w], idxv)
        pltpu.async_copy(tab_hbm.at[idxv.at[0]], buf0, sem0)

        @pl.loop(0, ch // 2)
        def _(i):
            j = i * 2
            pltpu.async_copy(tab_hbm.at[idxv.at[j + 1]], buf1, sem1)
            pltpu.make_async_copy(tab_hbm.at[idxv.at[j]], buf0, sem0).wait()
            pltpu.sync_copy(buf0, xs_hbm.at[pl.ds(base + j * _CHUNK, _CHUNK)])

            @pl.when(i < ch // 2 - 1)
            def _():
                pltpu.async_copy(tab_hbm.at[idxv.at[j + 2]], buf0, sem0)

            pltpu.make_async_copy(tab_hbm.at[idxv.at[j + 1]], buf1, sem1).wait()
            pltpu.sync_copy(
                buf1, xs_hbm.at[pl.ds(base + (j + 1) * _CHUNK, _CHUNK)])

    return gather(idx_r, table)


@functools.partial(jax.jit, static_argnums=(3,))
def _sc_scatter_add(idx_f, vals, zeros_nd, n):
    e_pad, d = vals.shape
    ew = e_pad // _NW
    ch = ew // _CHUNK
    rpt = n // _NS

    @functools.partial(
        pl.kernel,
        mesh=_sc_mesh(),
        compiler_params=pltpu.CompilerParams(use_tc_tiling_on_sc=False),
        out_type=jax.ShapeDtypeStruct((_NC * n, d), jnp.float32),
        scratch_types=[
            pltpu.VMEM((_CHUNK,), jnp.int32),
            pltpu.VMEM((_CHUNK,), jnp.int32),
            pltpu.VMEM((_CHUNK, d), jnp.float32),
            pltpu.VMEM((_CHUNK, d), jnp.float32),
            pltpu.VMEM_SHARED((n, d), jnp.float32),
            pltpu.SemaphoreType.DMA,
            pltpu.SemaphoreType.DMA,
        ],
    )
    def scatter(idx_hbm, val_hbm, zer_hbm, out_hbm, idx0, idx1, buf0, buf1,
                acc, sem0, sem1):
        cid = lax.axis_index("c")
        sid = lax.axis_index("s")
        w = sid * _NC + cid
        base = w * ew
        r0 = sid * rpt
        pltpu.sync_copy(zer_hbm.at[pl.ds(r0, rpt)], acc.at[pl.ds(r0, rpt)])
        plsc.subcore_barrier()

        @pl.loop(0, ch // 2)
        def _(i):
            j = i * 2

            @pl.when(i > 0)
            def _():
                pltpu.make_async_copy(buf0, acc.at[idx0], sem0).wait()
            pltpu.sync_copy(idx_hbm.at[pl.ds(base + j * _CHUNK, _CHUNK)], idx0)
            pltpu.sync_copy(val_hbm.at[pl.ds(base + j * _CHUNK, _CHUNK)], buf0)
            pltpu.async_copy(buf0, acc.at[idx0], sem0, add=True)

            @pl.when(i > 0)
            def _():
                pltpu.make_async_copy(buf1, acc.at[idx1], sem1).wait()
            pltpu.sync_copy(
                idx_hbm.at[pl.ds(base + (j + 1) * _CHUNK, _CHUNK)], idx1)
            pltpu.sync_copy(
                val_hbm.at[pl.ds(base + (j + 1) * _CHUNK, _CHUNK)], buf1)
            pltpu.async_copy(buf1, acc.at[idx1], sem1, add=True)

        pltpu.make_async_copy(buf0, acc.at[idx0], sem0).wait()
        pltpu.make_async_copy(buf1, acc.at[idx1], sem1).wait()
        plsc.subcore_barrier()
        pltpu.sync_copy(acc.at[pl.ds(r0, rpt)],
                        out_hbm.at[pl.ds(cid * n + r0, rpt)])

    return scatter(idx_f, vals, zeros_nd)


def _lin_relu2_body(x_ref, w_ref, b_ref, o_ref, ob_ref):
    r = jnp.maximum(
        jnp.dot(x_ref[...], w_ref[...], preferred_element_type=jnp.float32)
        + b_ref[...], 0.0)
    o_ref[...] = r
    ob_ref[...] = r.astype(jnp.bfloat16)


def _tc_lin_relu2(xx, w, b):
    n, dc = xx.shape[0], w.shape[1]
    return pl.pallas_call(
        _lin_relu2_body,
        out_shape=[jax.ShapeDtypeStruct((n, dc), jnp.float32),
                   jax.ShapeDtypeStruct((n, dc), jnp.bfloat16)],
    )(xx, w, b.reshape(1, -1))


def _hidden_body(x_ref, w_ref, b_ref, o_ref):
    o_ref[...] = jnp.maximum(
        jnp.dot(x_ref[...], w_ref[...], preferred_element_type=jnp.float32)
        + b_ref[...], 0.0).astype(jnp.bfloat16)


def _tc_hidden(ea_p, we1, be1, blk=8192):
    e_pad = ea_p.shape[0]
    din, dh = we1.shape
    return pl.pallas_call(
        _hidden_body,
        grid=(e_pad // blk,),
        in_specs=[
            pl.BlockSpec((blk, din), lambda i: (i, 0)),
            pl.BlockSpec((din, dh), lambda i: (0, 0)),
            pl.BlockSpec((1, dh), lambda i: (0, 0)),
        ],
        out_specs=pl.BlockSpec((blk, dh), lambda i: (i, 0)),
        out_shape=jax.ShapeDtypeStruct((e_pad, dh), jnp.bfloat16),
    )(ea_p, we1, be1.reshape(1, -1))


def _msg_body(e_total, blk, hb_ref, xs_ref, w2_ref, bmat_ref, rexp_ref,
              rsum_ref, o_ref):
    ewb = jnp.dot(hb_ref[...], w2_ref[...],
                  preferred_element_type=jnp.float32)
    xse = jnp.dot(xs_ref[...], rexp_ref[...],
                  preferred_element_type=jnp.float32)
    acc = (jnp.dot((ewb * xse).astype(jnp.bfloat16), rsum_ref[...],
                   preferred_element_type=jnp.float32)
           + jnp.dot(xs_ref[...], bmat_ref[...],
                     preferred_element_type=jnp.float32))
    rid = pl.program_id(0) * blk + lax.broadcasted_iota(jnp.int32, (blk, 1), 0)
    o_ref[...] = jnp.where(rid < e_total, acc, 0.0)


def _tc_msg(hidden, xs, we2, bmat, rexp, rsum, e_total, blk=2048):
    e_pad = hidden.shape[0]
    dh = hidden.shape[1]
    dc = xs.shape[1]
    return pl.pallas_call(
        functools.partial(_msg_body, e_total, blk),
        grid=(e_pad // blk,),
        in_specs=[
            pl.BlockSpec((blk, dh), lambda i: (i, 0)),
            pl.BlockSpec((blk, dc), lambda i: (i, 0)),
            pl.BlockSpec((dh, dc * dc), lambda i: (0, 0)),
            pl.BlockSpec((dc, dc), lambda i: (0, 0)),
            pl.BlockSpec((dc, dc * dc), lambda i: (0, 0)),
            pl.BlockSpec((dc * dc, dc), lambda i: (0, 0)),
        ],
        out_specs=pl.BlockSpec((blk, dc), lambda i: (i, 0)),
        out_shape=jax.ShapeDtypeStruct((e_pad, dc), jnp.float32),
    )(hidden, xs, we2, bmat, rexp, rsum)


def _update_body(n, dc, s_ref, aggp_ref, aggq_ref, degp_ref, wr_ref, bc_ref,
                 wih_ref, whh_ref, bih_ref, bhh_ref, o_ref, ob_ref):
    s = s_ref[...]
    aggp = aggp_ref[...]
    aggq = aggq_ref[...]
    degp = degp_ref[...]
    agg = (aggp[0:n] + aggp[n:]) + (aggq[0:n] + aggq[n:])
    deg = jnp.maximum(degp[0:n, 0:1] + degp[n:, 0:1], 1.0)
    m = jnp.maximum(
        jnp.dot(s, wr_ref[...], preferred_element_type=jnp.float32)
        + agg / deg + bc_ref[...], 0.0)
    gi = lax.dot_general(m, wih_ref[...], (((1,), (1,)), ((), ())),
                         preferred_element_type=jnp.float32) + bih_ref[...]
    gh = lax.dot_general(s, whh_ref[...], (((1,), (1,)), ((), ())),
                         preferred_element_type=jnp.float32) + bhh_ref[...]
    r = jax.nn.sigmoid(gi[:, 0:dc] + gh[:, 0:dc])
    z = jax.nn.sigmoid(gi[:, dc:2 * dc] + gh[:, dc:2 * dc])
    ng = jnp.tanh(gi[:, 2 * dc:3 * dc] + r * gh[:, 2 * dc:3 * dc])
    s_new = (1.0 - z) * ng + z * s
    o_ref[...] = s_new
    ob_ref[...] = s_new.astype(jnp.bfloat16)


def _tc_update(s, aggp, aggq, degp, wroot, bconv, gwih, gwhh, gbih, gbhh):
    n, dc = s.shape
    return pl.pallas_call(
        functools.partial(_update_body, n, dc),
        out_shape=[jax.ShapeDtypeStruct((n, dc), jnp.float32),
                   jax.ShapeDtypeStruct((n, dc), jnp.bfloat16)],
    )(s, aggp, aggq, degp, wroot, bconv.reshape(1, -1), gwih, gwhh,
      gbih.reshape(1, -1), gbhh.reshape(1, -1))


def _set2set_body(steps, ngr, n, dc, s_ref, batch_ref, ga_ref,
                  wih_ref, whh_ref, bih_ref, bhh_ref,
                  w1_ref, b1_ref, w2_ref, b2_ref, o_ref):
    s = s_ref[...]
    br = batch_ref[...]
    gidx = lax.broadcasted_iota(jnp.int32, (ngr, n), 0)
    msk = gidx == br
    qh = jnp.zeros((ngr, dc), jnp.float32)
    qc = jnp.zeros((ngr, dc), jnp.float32)
    q_star = jnp.zeros((ngr, 2 * dc), jnp.float32)
    for _ in range(steps):
        gates = (
            lax.dot_general(q_star, wih_ref[...], (((1,), (1,)), ((), ())),
                            preferred_element_type=jnp.float32)
            + bih_ref[...]
            + lax.dot_general(qh, whh_ref[...], (((1,), (1,)), ((), ())),
                              preferred_element_type=jnp.float32)
            + bhh_ref[...])
        ig = jax.nn.sigmoid(gates[:, 0:dc])
        fg = jax.nn.sigmoid(gates[:, dc:2 * dc])
        gg = jnp.tanh(gates[:, 2 * dc:3 * dc])
        og = jax.nn.sigmoid(gates[:, 3 * dc:4 * dc])
        qc = fg * qc + ig * gg
        qh = og * jnp.tanh(qc)
        qs = lax.dot_general(qh, s, (((1,), (1,)), ((), ())),
                             preferred_element_type=jnp.float32)
        eb = jnp.where(msk, qs, -jnp.inf)
        emax = jnp.max(eb, axis=1, keepdims=True)
        emax = jnp.where(emax > -jnp.inf, emax, 0.0)
        eeb = jnp.where(msk, jnp.exp(qs - emax), 0.0)
        den = jnp.sum(eeb, axis=1, keepdims=True)
        ab = eeb / (den + 1e-16)
        rvec = lax.dot_general(ab, s, (((1,), (0,)), ((), ())),
                               preferred_element_type=jnp.float32)
        q_star = jnp.concatenate([qh, rvec], axis=1)
    og2 = jnp.concatenate([q_star, ga_ref[...]], axis=1)
    h1 = jnp.maximum(
        lax.dot_general(og2, w1_ref[...], (((1,), (0,)), ((), ())),
                        preferred_element_type=jnp.float32) + b1_ref[...], 0.0)
    o_ref[...] = lax.dot_general(h1, w2_ref[...], (((1,), (0,)), ((), ())),
                                 preferred_element_type=jnp.float32) + b2_ref[...]


def _tc_set2set(s, batch_row, ga, lwih, lwhh, lbih, lbhh, w1, b1, w2, b2):
    n, dc = s.shape
    ngr = ga.shape[0]
    dout = w2.shape[1]
    return pl.pallas_call(
        functools.partial(_set2set_body, _STEPS, ngr, n, dc),
        out_shape=jax.ShapeDtypeStruct((ngr, dout), jnp.float32),
    )(s, batch_row, ga, lwih, lwhh, lbih.reshape(1, -1), lbhh.reshape(1, -1),
      w1, b1.reshape(1, -1), w2, b2.reshape(1, -1))


def kernel(x, edge_index, edge_attr, batch, graph_attr, W0, b0, We1, be1,
           We2, be2, Wroot, bconv, gru_Wih, gru_Whh, gru_bih, gru_bhh,
           lstm_Wih, lstm_Whh, lstm_bih, lstm_bhh, W1, b1, W2, b2):
    n = x.shape[0]
    e = edge_index.shape[1]
    dc = W0.shape[1]

    align = _NW * _CHUNK
    e_pad = ((e + align - 1) // align) * align
    ch = e_pad // _NW // _CHUNK
    pad = e_pad - e

    half = e_pad // 2
    chh = ch // 2
    src_p = jnp.pad(edge_index[0], (0, pad))
    dst_p = jnp.pad(edge_index[1], (0, pad))
    src_ra = src_p[:half].reshape(_NW, chh, _CHUNK)
    src_rb = src_p[half:].reshape(_NW, chh, _CHUNK)
    dst_a = dst_p[:half]
    dst_b = dst_p[half:]
    dst_f = dst_p
    ea_p = jnp.pad(edge_attr, ((0, pad), (0, 0)))
    ones8 = jnp.pad(jnp.ones((e, 8), jnp.float32), ((0, pad), (0, 0)))
    zer_dc = jnp.zeros((n, dc), jnp.float32)
    zer8 = jnp.zeros((n, 8), jnp.float32)
    batch_row = batch.reshape(1, n)

    s, s_bf = _tc_lin_relu2(x, W0, b0)
    hid_a = _tc_hidden(ea_p[:half], We1, be1)
    hid_b = _tc_hidden(ea_p[half:], We1, be1)
    degp = _sc_scatter_add(dst_f, ones8, zer8, n)
    we2b = We2.astype(jnp.bfloat16)
    e_a = min(e, half)
    e_b = e - e_a

    bmat = be2.reshape(dc, dc)
    lane = jnp.arange(dc * dc, dtype=jnp.int32)
    rexp = (lane[None, :] // dc == jnp.arange(dc, dtype=jnp.int32)[:, None]
            ).astype(jnp.bfloat16)
    rsum = (lane[:, None] % dc == jnp.arange(dc, dtype=jnp.int32)[None, :]
            ).astype(jnp.bfloat16)

    for _ in range(_MP_TIMES):
        xs_a = _sc_gather(src_ra, s_bf, half)
        xs_b = _sc_gather(src_rb, s_bf, half)
        msg_a = _tc_msg(hid_a, xs_a, we2b, bmat, rexp, rsum, e_a)
        agg_a = _sc_scatter_add(dst_a, msg_a, zer_dc, n)
        msg_b = _tc_msg(hid_b, xs_b, we2b, bmat, rexp, rsum, e_b)
        agg_b = _sc_scatter_add(dst_b, msg_b, zer_dc, n)
        s, s_bf = _tc_update(s, agg_a, agg_b, degp, Wroot, bconv,
                             gru_Wih, gru_Whh, gru_bih, gru_bhh)

    return _tc_set2set(s, batch_row, graph_attr, lstm_Wih, lstm_Whh,
                       lstm_bih, lstm_bhh, W1, b1, W2, b2)


# speedup vs baseline: 2.7736x; 1.0018x over previous
import functools

import jax
import jax.numpy as jnp
from jax import lax
from jax.experimental import pallas as pl
from jax.experimental.pallas import tpu as pltpu
from jax.experimental.pallas import tpu_sc as plsc

_NC = 2
_NS = 16
_NW = _NC * _NS
_CHUNK = 128

_MP_TIMES = 3
_STEPS = 3


def _sc_mesh():
    return plsc.VectorSubcoreMesh(core_axis_name="c", subcore_axis_name="s",
                                  num_cores=_NC, num_subcores=_NS)


@functools.partial(jax.jit, static_argnums=(2,))
def _sc_gather(idx_r, table, e_pad):
    n, d = table.shape
    ew = e_pad // _NW
    ch = ew // _CHUNK

    @functools.partial(
        pl.kernel,
        mesh=_sc_mesh(),
        compiler_params=pltpu.CompilerParams(use_tc_tiling_on_sc=False),
        out_type=jax.ShapeDtypeStruct((e_pad, d), jnp.bfloat16),
        scratch_types=[
            pltpu.VMEM((ch, _CHUNK), jnp.int32),
            pltpu.VMEM((_CHUNK, d), jnp.bfloat16),
            pltpu.VMEM((_CHUNK, d), jnp.bfloat16),
            pltpu.VMEM_SHARED((n, d), jnp.bfloat16),
            pltpu.SemaphoreType.DMA,
            pltpu.SemaphoreType.DMA,
        ],
    )
    def gather(idx_hbm, tab_hbm, xs_hbm, idxv, buf0, buf1, tabs, sem0, sem1):
        cid = lax.axis_index("c")
        sid = lax.axis_index("s")
        w = sid * _NC + cid
        base = w * ew
        rpt = n // _NS
        r0 = sid * rpt
        pltpu.sync_copy(tab_hbm.at[pl.ds(r0, rpt)], tabs.at[pl.ds(r0, rpt)])
        pltpu.sync_copy(idx_hbm.at[w], idxv)
        plsc.subcore_barrier()
        pltpu.async_copy(tabs.at[idxv.at[0]], buf0, sem0)

        @pl.loop(0, ch // 2)
        def _(i):
            j = i * 2
            pltpu.async_copy(tabs.at[idxv.at[j + 1]], buf1, sem1)
            pltpu.make_async_copy(tabs.at[idxv.at[j]], buf0, sem0).wait()
            pltpu.sync_copy(buf0, xs_hbm.at[pl.ds(base + j * _CHUNK, _CHUNK)])

            @pl.when(i < ch // 2 - 1)
            def _():
                pltpu.async_copy(tabs.at[idxv.at[j + 2]], buf0, sem0)

            pltpu.make_async_copy(tabs.at[idxv.at[j + 1]], buf1, sem1).wait()
            pltpu.sync_copy(
                buf1, xs_hbm.at[pl.ds(base + (j + 1) * _CHUNK, _CHUNK)])

    return gather(idx_r, table)


@functools.partial(jax.jit, static_argnums=(3,))
def _sc_scatter_add(idx_f, vals, zeros_nd, n):
    e_pad, d = vals.shape
    ew = e_pad // _NW
    ch = ew // _CHUNK
    rpt = n // _NS

    @functools.partial(
        pl.kernel,
        mesh=_sc_mesh(),
        compiler_params=pltpu.CompilerParams(use_tc_tiling_on_sc=False),
        out_type=jax.ShapeDtypeStruct((_NC * n, d), jnp.float32),
        scratch_types=[
            pltpu.VMEM((_CHUNK,), jnp.int32),
            pltpu.VMEM((_CHUNK,), jnp.int32),
            pltpu.VMEM((_CHUNK, d), jnp.float32),
            pltpu.VMEM((_CHUNK, d), jnp.float32),
            pltpu.VMEM_SHARED((n, d), jnp.float32),
            pltpu.SemaphoreType.DMA,
            pltpu.SemaphoreType.DMA,
        ],
    )
    def scatter(idx_hbm, val_hbm, zer_hbm, out_hbm, idx0, idx1, buf0, buf1,
                acc, sem0, sem1):
        cid = lax.axis_index("c")
        sid = lax.axis_index("s")
        w = sid * _NC + cid
        base = w * ew
        r0 = sid * rpt
        pltpu.sync_copy(zer_hbm.at[pl.ds(r0, rpt)], acc.at[pl.ds(r0, rpt)])
        plsc.subcore_barrier()

        @pl.loop(0, ch // 2)
        def _(i):
            j = i * 2

            @pl.when(i > 0)
            def _():
                pltpu.make_async_copy(buf0, acc.at[idx0], sem0).wait()
            pltpu.sync_copy(idx_hbm.at[pl.ds(base + j * _CHUNK, _CHUNK)], idx0)
            pltpu.sync_copy(val_hbm.at[pl.ds(base + j * _CHUNK, _CHUNK)], buf0)
            pltpu.async_copy(buf0, acc.at[idx0], sem0, add=True)

            @pl.when(i > 0)
            def _():
                pltpu.make_async_copy(buf1, acc.at[idx1], sem1).wait()
            pltpu.sync_copy(
                idx_hbm.at[pl.ds(base + (j + 1) * _CHUNK, _CHUNK)], idx1)
            pltpu.sync_copy(
                val_hbm.at[pl.ds(base + (j + 1) * _CHUNK, _CHUNK)], buf1)
            pltpu.async_copy(buf1, acc.at[idx1], sem1, add=True)

        pltpu.make_async_copy(buf0, acc.at[idx0], sem0).wait()
        pltpu.make_async_copy(buf1, acc.at[idx1], sem1).wait()
        plsc.subcore_barrier()
        pltpu.sync_copy(acc.at[pl.ds(r0, rpt)],
                        out_hbm.at[pl.ds(cid * n + r0, rpt)])

    return scatter(idx_f, vals, zeros_nd)


def _lin_relu2_body(x_ref, w_ref, b_ref, o_ref, ob_ref):
    r = jnp.maximum(
        jnp.dot(x_ref[...], w_ref[...], preferred_element_type=jnp.float32)
        + b_ref[...], 0.0)
    o_ref[...] = r
    ob_ref[...] = r.astype(jnp.bfloat16)


def _tc_lin_relu2(xx, w, b):
    n, dc = xx.shape[0], w.shape[1]
    return pl.pallas_call(
        _lin_relu2_body,
        out_shape=[jax.ShapeDtypeStruct((n, dc), jnp.float32),
                   jax.ShapeDtypeStruct((n, dc), jnp.bfloat16)],
    )(xx, w, b.reshape(1, -1))


def _hidden_body(x_ref, w_ref, b_ref, o_ref):
    o_ref[...] = jnp.maximum(
        jnp.dot(x_ref[...], w_ref[...], preferred_element_type=jnp.float32)
        + b_ref[...], 0.0).astype(jnp.bfloat16)


def _tc_hidden(ea_p, we1, be1, blk=8192):
    e_pad = ea_p.shape[0]
    din, dh = we1.shape
    return pl.pallas_call(
        _hidden_body,
        grid=(e_pad // blk,),
        in_specs=[
            pl.BlockSpec((blk, din), lambda i: (i, 0)),
            pl.BlockSpec((din, dh), lambda i: (0, 0)),
            pl.BlockSpec((1, dh), lambda i: (0, 0)),
        ],
        out_specs=pl.BlockSpec((blk, dh), lambda i: (i, 0)),
        out_shape=jax.ShapeDtypeStruct((e_pad, dh), jnp.bfloat16),
    )(ea_p, we1, be1.reshape(1, -1))


def _msg_body(e_total, blk, hb_ref, xs_ref, w2_ref, bmat_ref, rexp_ref,
              rsum_ref, o_ref):
    ewb = jnp.dot(hb_ref[...], w2_ref[...],
                  preferred_element_type=jnp.float32)
    xse = jnp.dot(xs_ref[...], rexp_ref[...],
                  preferred_element_type=jnp.float32)
    acc = (jnp.dot((ewb * xse).astype(jnp.bfloat16), rsum_ref[...],
                   preferred_element_type=jnp.float32)
           + jnp.dot(xs_ref[...], bmat_ref[...],
                     preferred_element_type=jnp.float32))
    rid = pl.program_id(0) * blk + lax.broadcasted_iota(jnp.int32, (blk, 1), 0)
    o_ref[...] = jnp.where(rid < e_total, acc, 0.0)


def _tc_msg(hidden, xs, we2, bmat, rexp, rsum, e_total, blk=2048):
    e_pad = hidden.shape[0]
    dh = hidden.shape[1]
    dc = xs.shape[1]
    return pl.pallas_call(
        functools.partial(_msg_body, e_total, blk),
        grid=(e_pad // blk,),
        in_specs=[
            pl.BlockSpec((blk, dh), lambda i: (i, 0)),
            pl.BlockSpec((blk, dc), lambda i: (i, 0)),
            pl.BlockSpec((dh, dc * dc), lambda i: (0, 0)),
            pl.BlockSpec((dc, dc), lambda i: (0, 0)),
            pl.BlockSpec((dc, dc * dc), lambda i: (0, 0)),
            pl.BlockSpec((dc * dc, dc), lambda i: (0, 0)),
        ],
        out_specs=pl.BlockSpec((blk, dc), lambda i: (i, 0)),
        out_shape=jax.ShapeDtypeStruct((e_pad, dc), jnp.float32),
    )(hidden, xs, we2, bmat, rexp, rsum)


def _update_body(n, dc, s_ref, aggp_ref, aggq_ref, degp_ref, wr_ref, bc_ref,
                 wih_ref, whh_ref, bih_ref, bhh_ref, o_ref, ob_ref):
    s = s_ref[...]
    aggp = aggp_ref[...]
    aggq = aggq_ref[...]
    degp = degp_ref[...]
    agg = (aggp[0:n] + aggp[n:]) + (aggq[0:n] + aggq[n:])
    deg = jnp.maximum(degp[0:n, 0:1] + degp[n:, 0:1], 1.0)
    m = jnp.maximum(
        jnp.dot(s, wr_ref[...], preferred_element_type=jnp.float32)
        + agg / deg + bc_ref[...], 0.0)
    gi = lax.dot_general(m, wih_ref[...], (((1,), (1,)), ((), ())),
                         preferred_element_type=jnp.float32) + bih_ref[...]
    gh = lax.dot_general(s, whh_ref[...], (((1,), (1,)), ((), ())),
                         preferred_element_type=jnp.float32) + bhh_ref[...]
    r = jax.nn.sigmoid(gi[:, 0:dc] + gh[:, 0:dc])
    z = jax.nn.sigmoid(gi[:, dc:2 * dc] + gh[:, dc:2 * dc])
    ng = jnp.tanh(gi[:, 2 * dc:3 * dc] + r * gh[:, 2 * dc:3 * dc])
    s_new = (1.0 - z) * ng + z * s
    o_ref[...] = s_new
    ob_ref[...] = s_new.astype(jnp.bfloat16)


def _tc_update(s, aggp, aggq, degp, wroot, bconv, gwih, gwhh, gbih, gbhh):
    n, dc = s.shape
    return pl.pallas_call(
        functools.partial(_update_body, n, dc),
        out_shape=[jax.ShapeDtypeStruct((n, dc), jnp.float32),
                   jax.ShapeDtypeStruct((n, dc), jnp.bfloat16)],
    )(s, aggp, aggq, degp, wroot, bconv.reshape(1, -1), gwih, gwhh,
      gbih.reshape(1, -1), gbhh.reshape(1, -1))


def _set2set_body(steps, ngr, n, dc, s_ref, batch_ref, ga_ref,
                  wih_ref, whh_ref, bih_ref, bhh_ref,
                  w1_ref, b1_ref, w2_ref, b2_ref, o_ref):
    s = s_ref[...]
    br = batch_ref[...]
    gidx = lax.broadcasted_iota(jnp.int32, (ngr, n), 0)
    msk = gidx == br
    qh = jnp.zeros((ngr, dc), jnp.float32)
    qc = jnp.zeros((ngr, dc), jnp.float32)
    q_star = jnp.zeros((ngr, 2 * dc), jnp.float32)
    for _ in range(steps):
        gates = (
            lax.dot_general(q_star, wih_ref[...], (((1,), (1,)), ((), ())),
                            preferred_element_type=jnp.float32)
            + bih_ref[...]
            + lax.dot_general(qh, whh_ref[...], (((1,), (1,)), ((), ())),
                              preferred_element_type=jnp.float32)
            + bhh_ref[...])
        ig = jax.nn.sigmoid(gates[:, 0:dc])
        fg = jax.nn.sigmoid(gates[:, dc:2 * dc])
        gg = jnp.tanh(gates[:, 2 * dc:3 * dc])
        og = jax.nn.sigmoid(gates[:, 3 * dc:4 * dc])
        qc = fg * qc + ig * gg
        qh = og * jnp.tanh(qc)
        qs = lax.dot_general(qh, s, (((1,), (1,)), ((), ())),
                             preferred_element_type=jnp.float32)
        eb = jnp.where(msk, qs, -jnp.inf)
        emax = jnp.max(eb, axis=1, keepdims=True)
        emax = jnp.where(emax > -jnp.inf, emax, 0.0)
        eeb = jnp.where(msk, jnp.exp(qs - emax), 0.0)
        den = jnp.sum(eeb, axis=1, keepdims=True)
        ab = eeb / (den + 1e-16)
        rvec = lax.dot_general(ab, s, (((1,), (0,)), ((), ())),
                               preferred_element_type=jnp.float32)
        q_star = jnp.concatenate([qh, rvec], axis=1)
    og2 = jnp.concatenate([q_star, ga_ref[...]], axis=1)
    h1 = jnp.maximum(
        lax.dot_general(og2, w1_ref[...], (((1,), (0,)), ((), ())),
                        preferred_element_type=jnp.float32) + b1_ref[...], 0.0)
    o_ref[...] = lax.dot_general(h1, w2_ref[...], (((1,), (0,)), ((), ())),
                                 preferred_element_type=jnp.float32) + b2_ref[...]


def _tc_set2set(s, batch_row, ga, lwih, lwhh, lbih, lbhh, w1, b1, w2, b2):
    n, dc = s.shape
    ngr = ga.shape[0]
    dout = w2.shape[1]
    return pl.pallas_call(
        functools.partial(_set2set_body, _STEPS, ngr, n, dc),
        out_shape=jax.ShapeDtypeStruct((ngr, dout), jnp.float32),
    )(s, batch_row, ga, lwih, lwhh, lbih.reshape(1, -1), lbhh.reshape(1, -1),
      w1, b1.reshape(1, -1), w2, b2.reshape(1, -1))


def kernel(x, edge_index, edge_attr, batch, graph_attr, W0, b0, We1, be1,
           We2, be2, Wroot, bconv, gru_Wih, gru_Whh, gru_bih, gru_bhh,
           lstm_Wih, lstm_Whh, lstm_bih, lstm_bhh, W1, b1, W2, b2):
    n = x.shape[0]
    e = edge_index.shape[1]
    dc = W0.shape[1]

    align = _NW * _CHUNK
    e_pad = ((e + align - 1) // align) * align
    ch = e_pad // _NW // _CHUNK
    pad = e_pad - e

    half = e_pad // 2
    chh = ch // 2
    src_p = jnp.pad(edge_index[0], (0, pad))
    dst_p = jnp.pad(edge_index[1], (0, pad))
    src_ra = src_p[:half].reshape(_NW, chh, _CHUNK)
    src_rb = src_p[half:].reshape(_NW, chh, _CHUNK)
    dst_a = dst_p[:half]
    dst_b = dst_p[half:]
    dst_f = dst_p
    ea_p = jnp.pad(edge_attr, ((0, pad), (0, 0)))
    ones1 = jnp.pad(jnp.ones((e, 1), jnp.float32), ((0, pad), (0, 0)))
    zer_dc = jnp.zeros((n, dc), jnp.float32)
    zer1 = jnp.zeros((n, 1), jnp.float32)
    batch_row = batch.reshape(1, n)

    s, s_bf = _tc_lin_relu2(x, W0, b0)
    hid_a = _tc_hidden(ea_p[:half], We1, be1)
    hid_b = _tc_hidden(ea_p[half:], We1, be1)
    degp = _sc_scatter_add(dst_f, ones1, zer1, n)
    we2b = We2.astype(jnp.bfloat16)
    e_a = min(e, half)
    e_b = e - e_a

    bmat = be2.reshape(dc, dc)
    lane = jnp.arange(dc * dc, dtype=jnp.int32)
    rexp = (lane[None, :] // dc == jnp.arange(dc, dtype=jnp.int32)[:, None]
            ).astype(jnp.bfloat16)
    rsum = (lane[:, None] % dc == jnp.arange(dc, dtype=jnp.int32)[None, :]
            ).astype(jnp.bfloat16)

    for _ in range(_MP_TIMES):
        xs_a = _sc_gather(src_ra, s_bf, half)
        xs_b = _sc_gather(src_rb, s_bf, half)
        msg_a = _tc_msg(hid_a, xs_a, we2b, bmat, rexp, rsum, e_a)
        agg_a = _sc_scatter_add(dst_a, msg_a, zer_dc, n)
        msg_b = _tc_msg(hid_b, xs_b, we2b, bmat, rexp, rsum, e_b)
        agg_b = _sc_scatter_add(dst_b, msg_b, zer_dc, n)
        s, s_bf = _tc_update(s, agg_a, agg_b, degp, Wroot, bconv,
                             gru_Wih, gru_Whh, gru_bih, gru_bhh)

    return _tc_set2set(s, batch_row, graph_attr, lstm_Wih, lstm_Whh,
                       lstm_bih, lstm_bhh, W1, b1, W2, b2)
